# baseline (speedup 1.0000x reference)
import jax
import jax.numpy as jnp
from jax import lax
from jax.experimental import pallas as pl
from jax.experimental.pallas import tpu as pltpu
from jax.experimental.pallas import tpu_sc as plsc

NUM_EMBEDDINGS = 1000000
EMBEDDING_DIM = 32
BATCH = 16384
FIELDS = 26

_B = BATCH * FIELDS
_NW = 32
_RW = BATCH // _NW
_PER_W = _B // _NW
_NBUF = 2
_NCHUNK = 32
_CHUNK = _PER_W // _NCHUNK
_NGRP = _CHUNK // 16
_MAGIC = (1 << 20) // FIELDS + 1


def _body(table_hbm, idxp_hbm, out_hbm, idxp_v, list0, list1,
          rows0, rows1, *sems):
    nc = 2
    wid = lax.axis_index("s") * nc + lax.axis_index("c")
    base = wid * _PER_W
    lists = (list0, list1)
    rows = (rows0, rows1)
    gsem = sems[:_NBUF]
    ssem = sems[_NBUF:]

    iota = lax.broadcasted_iota(jnp.int32, (16,), 0)

    def compact(c, slot):
        @pl.loop(0, _NGRP)
        def _(g):
            e = iota + c * _CHUNK + g * 16
            r = (e * _MAGIC) >> 20
            f = e - r * FIELDS
            v = plsc.load_gather(idxp_v, [r, f])
            plsc.store_scatter(lists[slot], [iota + g * 16], v)

    def gather(slot):
        return pltpu.async_copy(
            table_hbm.at[lists[slot]], rows[slot], gsem[slot])

    def store(c, slot):
        return pltpu.async_copy(
            rows[slot],
            out_hbm.at[pl.ds(base + c * _CHUNK, _CHUNK)], ssem[slot])

    pltpu.sync_copy(idxp_hbm.at[pl.ds(wid * _RW, _RW)], idxp_v)

    depth = _NBUF - 1
    pending_g = [None] * _NBUF
    pending_s = [None] * _NBUF
    for c in range(depth):
        compact(c, c % _NBUF)
        pending_g[c % _NBUF] = gather(c % _NBUF)
    for c in range(_NCHUNK):
        slot = c % _NBUF
        n = c + depth
        pending_g[slot].wait()
        pending_g[slot] = None
        pending_s[slot] = store(c, slot)
        if n < _NCHUNK:
            s2 = n % _NBUF
            if pending_s[s2] is not None:
                pending_s[s2].wait()
                pending_s[s2] = None
            compact(n, s2)
            pending_g[s2] = gather(s2)
    for s in pending_s:
        if s is not None:
            s.wait()


@jax.jit
def _embed(idxp, weight):
    mesh = plsc.VectorSubcoreMesh(core_axis_name="c", subcore_axis_name="s")
    fn = pl.kernel(
        _body,
        out_type=jax.ShapeDtypeStruct((_B, EMBEDDING_DIM), jnp.float32),
        mesh=mesh,
        scratch_types=[
            pltpu.VMEM((_RW, 128), jnp.int32),
            pltpu.VMEM((_CHUNK,), jnp.int32),
            pltpu.VMEM((_CHUNK,), jnp.int32),
            pltpu.VMEM((_CHUNK, EMBEDDING_DIM), jnp.float32),
            pltpu.VMEM((_CHUNK, EMBEDDING_DIM), jnp.float32),
        ] + [pltpu.SemaphoreType.DMA] * (2 * _NBUF),
        compiler_params=pltpu.CompilerParams(
            use_tc_tiling_on_sc=False, needs_layout_passes=False),
    )
    return fn(weight, idxp)


def kernel(input, weight):
    idxp = jnp.pad(input, ((0, 0), (0, 128 - FIELDS)))
    out = _embed(idxp, weight)
    return out.reshape(BATCH, FIELDS, EMBEDDING_DIM)


# speedup vs baseline: 1.0260x; 1.0260x over previous
import jax
import jax.numpy as jnp
from jax import lax
from jax.experimental import pallas as pl
from jax.experimental.pallas import tpu as pltpu
from jax.experimental.pallas import tpu_sc as plsc

NUM_EMBEDDINGS = 1000000
EMBEDDING_DIM = 32
BATCH = 16384
FIELDS = 26

_B = BATCH * FIELDS
_NW = 32
_PER_W = _B // _NW
_NBUF = 8
_NCHUNK = 32
_CHUNK = _PER_W // _NCHUNK


def _body(table_hbm, idx_hbm, out_hbm, idx_v, rows_v, *sems):
    nc = 2
    wid = lax.axis_index("s") * nc + lax.axis_index("c")
    base = wid * _PER_W
    gsem = sems[:_NBUF]
    ssem = sems[_NBUF:]

    def gather(c, buf):
        return pltpu.async_copy(
            table_hbm.at[idx_v.at[pl.ds(c * _CHUNK, _CHUNK)]],
            rows_v.at[buf], gsem[buf])

    def store(c, buf):
        return pltpu.async_copy(
            rows_v.at[buf],
            out_hbm.at[pl.ds(base + c * _CHUNK, _CHUNK)], ssem[buf])

    pltpu.sync_copy(idx_hbm.at[pl.ds(base, _PER_W)], idx_v)

    depth = _NBUF - 1
    pending_g = [None] * _NBUF
    pending_s = [None] * _NBUF
    for c in range(depth):
        pending_g[c % _NBUF] = gather(c, c % _NBUF)
    for c in range(_NCHUNK):
        buf = c % _NBUF
        pending_g[buf].wait()
        pending_g[buf] = None
        pending_s[buf] = store(c, buf)
        n = c + depth
        if n < _NCHUNK:
            b2 = n % _NBUF
            if pending_s[b2] is not None:
                pending_s[b2].wait()
                pending_s[b2] = None
            pending_g[b2] = gather(n, b2)
    for s in pending_s:
        if s is not None:
            s.wait()


@jax.jit
def _embed(idx_flat, weight):
    mesh = plsc.VectorSubcoreMesh(core_axis_name="c", subcore_axis_name="s")
    fn = pl.kernel(
        _body,
        out_type=jax.ShapeDtypeStruct((_B, EMBEDDING_DIM), jnp.float32),
        mesh=mesh,
        scratch_types=[
            pltpu.VMEM((_PER_W,), jnp.int32),
            pltpu.VMEM((_NBUF, _CHUNK, EMBEDDING_DIM), jnp.float32),
        ] + [pltpu.SemaphoreType.DMA] * (2 * _NBUF),
        compiler_params=pltpu.CompilerParams(use_tc_tiling_on_sc=False),
    )
    return fn(weight, idx_flat)


def kernel(input, weight):
    out = _embed(input.reshape(-1), weight)
    return out.reshape(BATCH, FIELDS, EMBEDDING_DIM)
